# Initial kernel scaffold; baseline (speedup 1.0000x reference)
import functools

import jax
import jax.numpy as jnp
from jax import lax
from jax.experimental import pallas as pl
from jax.experimental.pallas import tpu as pltpu
from jax.experimental.pallas import tpu_sc as plsc

NUM_CORES = 2
NUM_SUBCORES = 16
LANES = 16
NUM_WORKERS = NUM_CORES * NUM_SUBCORES
CHUNK = 80


def _dot_head_kernel(d, n_chunks, table_hbm, src_idx_hbm, tgt_idx_hbm, out_hbm,
                     sidx_v, tidx_v, src_v, tgt_v, out_v, sem0, sem1):
    wid = lax.axis_index("s") * NUM_CORES + lax.axis_index("c")
    base = wid * (n_chunks * CHUNK)
    inv_d = 1.0 / d

    @pl.loop(0, n_chunks)
    def _chunk(j):
        off = base + j * CHUNK
        pltpu.sync_copy(src_idx_hbm.at[pl.ds(off, CHUNK)], sidx_v)
        pltpu.sync_copy(tgt_idx_hbm.at[pl.ds(off, CHUNK)], tidx_v)
        c0 = pltpu.async_copy(table_hbm.at[sidx_v], src_v, sem0)
        c1 = pltpu.async_copy(table_hbm.at[tidx_v], tgt_v, sem1)
        c0.wait()
        c1.wait()

        @pl.loop(0, CHUNK)
        def _edge(e):
            acc = src_v[e, pl.ds(0, LANES)] * tgt_v[e, pl.ds(0, LANES)]
            for k in range(1, d // LANES):
                acc += (src_v[e, pl.ds(k * LANES, LANES)]
                        * tgt_v[e, pl.ds(k * LANES, LANES)])
            out_v[e] = jnp.sum(acc) * inv_d

        pltpu.sync_copy(out_v, out_hbm.at[pl.ds(off, CHUNK)])


def kernel(node_embeddings, edge_index):
    n, d = node_embeddings.shape
    b = edge_index.shape[1]
    assert d % LANES == 0
    assert b % (NUM_WORKERS * CHUNK) == 0
    n_chunks = b // (NUM_WORKERS * CHUNK)

    edge_index = edge_index.astype(jnp.int32)
    src_idx = edge_index[0]
    tgt_idx = edge_index[1]

    mesh = plsc.VectorSubcoreMesh(core_axis_name="c", subcore_axis_name="s")
    run = pl.kernel(
        functools.partial(_dot_head_kernel, d, n_chunks),
        out_type=jax.ShapeDtypeStruct((b,), jnp.float32),
        mesh=mesh,
        scratch_types=[
            pltpu.VMEM((CHUNK,), jnp.int32),
            pltpu.VMEM((CHUNK,), jnp.int32),
            pltpu.VMEM((CHUNK, d), jnp.float32),
            pltpu.VMEM((CHUNK, d), jnp.float32),
            pltpu.VMEM((CHUNK,), jnp.float32),
            pltpu.SemaphoreType.DMA,
            pltpu.SemaphoreType.DMA,
        ],
    )
    return run(node_embeddings, src_idx, tgt_idx)


# speedup vs baseline: 3.5521x; 3.5521x over previous
import functools

import jax
import jax.numpy as jnp
from jax import lax
from jax.experimental import pallas as pl
from jax.experimental.pallas import tpu as pltpu
from jax.experimental.pallas import tpu_sc as plsc

NUM_CORES = 2
NUM_SUBCORES = 16
LANES = 16
NUM_WORKERS = NUM_CORES * NUM_SUBCORES
CHUNK = 80


def _dot_head_kernel(d, n_chunks, table_hbm, src_idx_hbm, tgt_idx_hbm, out_hbm,
                     sidx_v, tidx_v, src_v, tgt_v, part_v, out_v, sem0, sem1):
    wid = lax.axis_index("s") * NUM_CORES + lax.axis_index("c")
    base = wid * (n_chunks * CHUNK)
    inv_d = 1.0 / d
    lane = lax.iota(jnp.int32, LANES)

    @pl.loop(0, n_chunks)
    def _chunk(j):
        off = base + j * CHUNK
        pltpu.sync_copy(src_idx_hbm.at[pl.ds(off, CHUNK)], sidx_v)
        pltpu.sync_copy(tgt_idx_hbm.at[pl.ds(off, CHUNK)], tidx_v)
        c0 = pltpu.async_copy(table_hbm.at[sidx_v], src_v, sem0)
        c1 = pltpu.async_copy(table_hbm.at[tidx_v], tgt_v, sem1)
        c0.wait()
        c1.wait()

        @pl.loop(0, CHUNK)
        def _edge(e):
            acc = src_v[e, pl.ds(0, LANES)] * tgt_v[e, pl.ds(0, LANES)]
            for k in range(1, d // LANES):
                acc += (src_v[e, pl.ds(k * LANES, LANES)]
                        * tgt_v[e, pl.ds(k * LANES, LANES)])
            part_v[e, :] = acc

        for g in range(CHUNK // LANES):
            row = g * LANES + lane
            tot = plsc.load_gather(part_v, [row, jnp.zeros_like(lane)])
            for c in range(1, LANES):
                tot += plsc.load_gather(part_v, [row, jnp.full_like(lane, c)])
            out_v[pl.ds(g * LANES, LANES)] = tot * inv_d

        pltpu.sync_copy(out_v, out_hbm.at[pl.ds(off, CHUNK)])


def kernel(node_embeddings, edge_index):
    n, d = node_embeddings.shape
    b = edge_index.shape[1]
    assert d % LANES == 0
    assert b % (NUM_WORKERS * CHUNK) == 0
    n_chunks = b // (NUM_WORKERS * CHUNK)

    edge_index = edge_index.astype(jnp.int32)
    src_idx = edge_index[0]
    tgt_idx = edge_index[1]

    mesh = plsc.VectorSubcoreMesh(core_axis_name="c", subcore_axis_name="s")
    run = pl.kernel(
        functools.partial(_dot_head_kernel, d, n_chunks),
        out_type=jax.ShapeDtypeStruct((b,), jnp.float32),
        mesh=mesh,
        compiler_params=pltpu.CompilerParams(needs_layout_passes=False),
        scratch_types=[
            pltpu.VMEM((CHUNK,), jnp.int32),
            pltpu.VMEM((CHUNK,), jnp.int32),
            pltpu.VMEM((CHUNK, d), jnp.float32),
            pltpu.VMEM((CHUNK, d), jnp.float32),
            pltpu.VMEM((CHUNK, LANES), jnp.float32),
            pltpu.VMEM((CHUNK,), jnp.float32),
            pltpu.SemaphoreType.DMA,
            pltpu.SemaphoreType.DMA,
        ],
    )
    return run(node_embeddings, src_idx, tgt_idx)


# speedup vs baseline: 5.4539x; 1.5354x over previous
import functools

import jax
import jax.numpy as jnp
from jax import lax
from jax.experimental import pallas as pl
from jax.experimental.pallas import tpu as pltpu
from jax.experimental.pallas import tpu_sc as plsc

NUM_CORES = 2
NUM_SUBCORES = 16
LANES = 16
NUM_WORKERS = NUM_CORES * NUM_SUBCORES
CHUNK = 80


def _dot_head_kernel(d, n_chunks, table_hbm, src_idx_hbm, tgt_idx_hbm, out_hbm,
                     sidx_v, tidx_v, src_v, tgt_v, part_v, out_v,
                     gsem0, gsem1, osem0, osem1):
    wid = lax.axis_index("s") * NUM_CORES + lax.axis_index("c")
    base = wid * (n_chunks * CHUNK)
    inv_d = 1.0 / d
    lane = lax.iota(jnp.int32, LANES)
    gsem = (gsem0, gsem1)
    osem = (osem0, osem1)

    def issue(jv, p):
        off = base + jv * CHUNK
        pltpu.sync_copy(src_idx_hbm.at[pl.ds(off, CHUNK)], sidx_v.at[p])
        pltpu.sync_copy(tgt_idx_hbm.at[pl.ds(off, CHUNK)], tidx_v.at[p])
        pltpu.async_copy(table_hbm.at[sidx_v.at[p]], src_v.at[p], gsem[p])
        pltpu.async_copy(table_hbm.at[tidx_v.at[p]], tgt_v.at[p], gsem[p])

    def wait_gathers(p):
        pltpu.make_async_copy(table_hbm.at[sidx_v.at[p]], src_v.at[p],
                              gsem[p]).wait()
        pltpu.make_async_copy(table_hbm.at[tidx_v.at[p]], tgt_v.at[p],
                              gsem[p]).wait()

    def wait_out(jv, p):
        off = base + (jv - 2) * CHUNK
        pltpu.make_async_copy(out_v.at[p], out_hbm.at[pl.ds(off, CHUNK)],
                              osem[p]).wait()

    def compute(jv, p):
        src_b = src_v.at[p]
        tgt_b = tgt_v.at[p]

        @pl.loop(0, CHUNK)
        def _edge(e):
            acc = src_b[e, pl.ds(0, LANES)] * tgt_b[e, pl.ds(0, LANES)]
            for k in range(1, d // LANES):
                acc += (src_b[e, pl.ds(k * LANES, LANES)]
                        * tgt_b[e, pl.ds(k * LANES, LANES)])
            part_v[e, :] = acc

        for g in range(CHUNK // LANES):
            row = g * LANES + lane
            tot = plsc.load_gather(part_v, [row, jnp.zeros_like(lane)])
            for c in range(1, LANES):
                tot += plsc.load_gather(part_v, [row, jnp.full_like(lane, c)])
            out_v[p, pl.ds(g * LANES, LANES)] = tot * inv_d

        off = base + jv * CHUNK
        pltpu.async_copy(out_v.at[p], out_hbm.at[pl.ds(off, CHUNK)], osem[p])

    def stage(jv, p):
        issue(jv + 1, 1 - p)
        wait_gathers(p)

        @pl.when(jv >= 2)
        def _():
            wait_out(jv, p)

        compute(jv, p)

    issue(0, 0)

    @pl.loop(0, n_chunks - 1, step=2)
    def _pair(j):
        stage(j, 0)
        stage(j + 1, 1)

    jl = n_chunks - 1
    wait_gathers(0)
    wait_out(jl, 0)
    compute(jl, 0)
    wait_out(jl + 1, 1)
    wait_out(jl + 2, 0)


def kernel(node_embeddings, edge_index):
    n, d = node_embeddings.shape
    b = edge_index.shape[1]
    assert d % LANES == 0
    assert b % (NUM_WORKERS * CHUNK) == 0
    n_chunks = b // (NUM_WORKERS * CHUNK)
    assert n_chunks % 2 == 1

    edge_index = edge_index.astype(jnp.int32)
    src_idx = edge_index[0]
    tgt_idx = edge_index[1]

    mesh = plsc.VectorSubcoreMesh(core_axis_name="c", subcore_axis_name="s")
    run = pl.kernel(
        functools.partial(_dot_head_kernel, d, n_chunks),
        out_type=jax.ShapeDtypeStruct((b,), jnp.float32),
        mesh=mesh,
        compiler_params=pltpu.CompilerParams(needs_layout_passes=False),
        scratch_types=[
            pltpu.VMEM((2, CHUNK), jnp.int32),
            pltpu.VMEM((2, CHUNK), jnp.int32),
            pltpu.VMEM((2, CHUNK, d), jnp.float32),
            pltpu.VMEM((2, CHUNK, d), jnp.float32),
            pltpu.VMEM((CHUNK, LANES), jnp.float32),
            pltpu.VMEM((2, CHUNK), jnp.float32),
            pltpu.SemaphoreType.DMA,
            pltpu.SemaphoreType.DMA,
            pltpu.SemaphoreType.DMA,
            pltpu.SemaphoreType.DMA,
        ],
    )
    return run(node_embeddings, src_idx, tgt_idx)


# speedup vs baseline: 7.6759x; 1.4074x over previous
import functools

import jax
import jax.numpy as jnp
from jax import lax
from jax.experimental import pallas as pl
from jax.experimental.pallas import tpu as pltpu
from jax.experimental.pallas import tpu_sc as plsc

NUM_CORES = 2
NUM_SUBCORES = 16
LANES = 16
NUM_WORKERS = NUM_CORES * NUM_SUBCORES
CHUNK = 80
UNROLL = 4


def _dot_head_kernel(d, n_chunks, table_hbm, src_idx_hbm, tgt_idx_hbm, out_hbm,
                     sidx_v, tidx_v, src_v, tgt_v, part_v, out_v,
                     isem0, isem1, gsem0, gsem1, osem0, osem1):
    wid = lax.axis_index("s") * NUM_CORES + lax.axis_index("c")
    base = wid * (n_chunks * CHUNK)
    inv_d = 1.0 / d
    lane = lax.iota(jnp.int32, LANES)
    isem = (isem0, isem1)
    gsem = (gsem0, gsem1)
    osem = (osem0, osem1)

    def issue_idx(jv, p):
        off = base + jv * CHUNK
        pltpu.async_copy(src_idx_hbm.at[pl.ds(off, CHUNK)], sidx_v.at[p],
                         isem[p])
        pltpu.async_copy(tgt_idx_hbm.at[pl.ds(off, CHUNK)], tidx_v.at[p],
                         isem[p])

    def wait_idx(p):
        pltpu.make_async_copy(src_idx_hbm.at[pl.ds(base, CHUNK)],
                              sidx_v.at[p], isem[p]).wait()
        pltpu.make_async_copy(tgt_idx_hbm.at[pl.ds(base, CHUNK)],
                              tidx_v.at[p], isem[p]).wait()

    def issue_gathers(p):
        pltpu.async_copy(table_hbm.at[sidx_v.at[p]], src_v.at[p], gsem[p])
        pltpu.async_copy(table_hbm.at[tidx_v.at[p]], tgt_v.at[p], gsem[p])

    def wait_gathers(p):
        pltpu.make_async_copy(table_hbm.at[sidx_v.at[p]], src_v.at[p],
                              gsem[p]).wait()
        pltpu.make_async_copy(table_hbm.at[tidx_v.at[p]], tgt_v.at[p],
                              gsem[p]).wait()

    def wait_out(p):
        pltpu.make_async_copy(out_v.at[p], out_hbm.at[pl.ds(base, CHUNK)],
                              osem[p]).wait()

    def compute(jv, p):
        src_b = src_v.at[p]
        tgt_b = tgt_v.at[p]

        @plsc.parallel_loop(0, CHUNK, unroll=UNROLL)
        def _edge(e):
            acc = src_b[e, pl.ds(0, LANES)] * tgt_b[e, pl.ds(0, LANES)]
            for k in range(1, d // LANES):
                acc += (src_b[e, pl.ds(k * LANES, LANES)]
                        * tgt_b[e, pl.ds(k * LANES, LANES)])
            part_v[e, :] = acc

        for g in range(CHUNK // LANES):
            row = g * LANES + lane
            tot = plsc.load_gather(part_v, [row, jnp.zeros_like(lane)])
            for c in range(1, LANES):
                tot += plsc.load_gather(part_v, [row, jnp.full_like(lane, c)])
            out_v[p, pl.ds(g * LANES, LANES)] = tot * inv_d

        off = base + jv * CHUNK
        pltpu.async_copy(out_v.at[p], out_hbm.at[pl.ds(off, CHUNK)], osem[p])

    def stage(jv, p):
        wait_idx(1 - p)
        issue_gathers(1 - p)
        wait_gathers(p)

        @pl.when(jv + 2 < n_chunks)
        def _():
            issue_idx(jv + 2, p)

        @pl.when(jv >= 2)
        def _():
            wait_out(p)

        compute(jv, p)

    issue_idx(0, 0)
    wait_idx(0)
    issue_gathers(0)
    issue_idx(1, 1)

    @pl.loop(0, n_chunks - 1, step=2)
    def _pair(j):
        stage(j, 0)
        stage(j + 1, 1)

    wait_gathers(0)
    wait_out(0)
    compute(n_chunks - 1, 0)
    wait_out(1)
    wait_out(0)


def kernel(node_embeddings, edge_index):
    n, d = node_embeddings.shape
    b = edge_index.shape[1]
    assert d % LANES == 0
    assert b % (NUM_WORKERS * CHUNK) == 0
    n_chunks = b // (NUM_WORKERS * CHUNK)
    assert n_chunks % 2 == 1 and n_chunks >= 3

    edge_index = edge_index.astype(jnp.int32)
    src_idx = edge_index[0]
    tgt_idx = edge_index[1]

    mesh = plsc.VectorSubcoreMesh(core_axis_name="c", subcore_axis_name="s")
    run = pl.kernel(
        functools.partial(_dot_head_kernel, d, n_chunks),
        out_type=jax.ShapeDtypeStruct((b,), jnp.float32),
        mesh=mesh,
        compiler_params=pltpu.CompilerParams(needs_layout_passes=False),
        scratch_types=[
            pltpu.VMEM((2, CHUNK), jnp.int32),
            pltpu.VMEM((2, CHUNK), jnp.int32),
            pltpu.VMEM((2, CHUNK, d), jnp.float32),
            pltpu.VMEM((2, CHUNK, d), jnp.float32),
            pltpu.VMEM((CHUNK, LANES), jnp.float32),
            pltpu.VMEM((2, CHUNK), jnp.float32),
            pltpu.SemaphoreType.DMA,
            pltpu.SemaphoreType.DMA,
            pltpu.SemaphoreType.DMA,
            pltpu.SemaphoreType.DMA,
            pltpu.SemaphoreType.DMA,
            pltpu.SemaphoreType.DMA,
        ],
    )
    return run(node_embeddings, src_idx, tgt_idx)


# speedup vs baseline: 7.6913x; 1.0020x over previous
import functools

import jax
import jax.numpy as jnp
from jax import lax
from jax.experimental import pallas as pl
from jax.experimental.pallas import tpu as pltpu
from jax.experimental.pallas import tpu_sc as plsc

NUM_CORES = 2
NUM_SUBCORES = 16
LANES = 16
NUM_WORKERS = NUM_CORES * NUM_SUBCORES
CHUNK = 80
UNROLL = 8


def _dot_head_kernel(d, n_chunks, table_hbm, src_idx_hbm, tgt_idx_hbm, out_hbm,
                     sidx_v, tidx_v, src_v, tgt_v, part_v, out_v,
                     isem0, isem1, gsem0, gsem1, osem0, osem1):
    wid = lax.axis_index("s") * NUM_CORES + lax.axis_index("c")
    base = wid * (n_chunks * CHUNK)
    inv_d = 1.0 / d
    lane = lax.iota(jnp.int32, LANES)
    isem = (isem0, isem1)
    gsem = (gsem0, gsem1)
    osem = (osem0, osem1)

    def issue_idx(jv, p):
        off = base + jv * CHUNK
        pltpu.async_copy(src_idx_hbm.at[pl.ds(off, CHUNK)], sidx_v.at[p],
                         isem[p])
        pltpu.async_copy(tgt_idx_hbm.at[pl.ds(off, CHUNK)], tidx_v.at[p],
                         isem[p])

    def wait_idx(p):
        pltpu.make_async_copy(src_idx_hbm.at[pl.ds(base, CHUNK)],
                              sidx_v.at[p], isem[p]).wait()
        pltpu.make_async_copy(tgt_idx_hbm.at[pl.ds(base, CHUNK)],
                              tidx_v.at[p], isem[p]).wait()

    def issue_gathers(p):
        pltpu.async_copy(table_hbm.at[sidx_v.at[p]], src_v.at[p], gsem[p])
        pltpu.async_copy(table_hbm.at[tidx_v.at[p]], tgt_v.at[p], gsem[p])

    def wait_gathers(p):
        pltpu.make_async_copy(table_hbm.at[sidx_v.at[p]], src_v.at[p],
                              gsem[p]).wait()
        pltpu.make_async_copy(table_hbm.at[tidx_v.at[p]], tgt_v.at[p],
                              gsem[p]).wait()

    def wait_out(p):
        pltpu.make_async_copy(out_v.at[p], out_hbm.at[pl.ds(base, CHUNK)],
                              osem[p]).wait()

    def compute(jv, p):
        src_b = src_v.at[p]
        tgt_b = tgt_v.at[p]

        @plsc.parallel_loop(0, CHUNK, unroll=UNROLL)
        def _edge(e):
            acc = src_b[e, pl.ds(0, LANES)] * tgt_b[e, pl.ds(0, LANES)]
            for k in range(1, d // LANES):
                acc += (src_b[e, pl.ds(k * LANES, LANES)]
                        * tgt_b[e, pl.ds(k * LANES, LANES)])
            part_v[e, :] = acc

        for g in range(CHUNK // LANES):
            row = g * LANES + lane
            tot = plsc.load_gather(part_v, [row, jnp.zeros_like(lane)])
            for c in range(1, LANES):
                tot += plsc.load_gather(part_v, [row, jnp.full_like(lane, c)])
            out_v[p, pl.ds(g * LANES, LANES)] = tot * inv_d

        off = base + jv * CHUNK
        pltpu.async_copy(out_v.at[p], out_hbm.at[pl.ds(off, CHUNK)], osem[p])

    def stage(jv, p):
        wait_idx(1 - p)
        issue_gathers(1 - p)
        wait_gathers(p)

        @pl.when(jv + 2 < n_chunks)
        def _():
            issue_idx(jv + 2, p)

        @pl.when(jv >= 2)
        def _():
            wait_out(p)

        compute(jv, p)

    issue_idx(0, 0)
    wait_idx(0)
    issue_gathers(0)
    issue_idx(1, 1)

    @pl.loop(0, n_chunks - 1, step=2)
    def _pair(j):
        stage(j, 0)
        stage(j + 1, 1)

    wait_gathers(0)
    wait_out(0)
    compute(n_chunks - 1, 0)
    wait_out(1)
    wait_out(0)


def kernel(node_embeddings, edge_index):
    n, d = node_embeddings.shape
    b = edge_index.shape[1]
    assert d % LANES == 0
    assert b % (NUM_WORKERS * CHUNK) == 0
    n_chunks = b // (NUM_WORKERS * CHUNK)
    assert n_chunks % 2 == 1 and n_chunks >= 3

    edge_index = edge_index.astype(jnp.int32)
    src_idx = edge_index[0]
    tgt_idx = edge_index[1]

    mesh = plsc.VectorSubcoreMesh(core_axis_name="c", subcore_axis_name="s")
    run = pl.kernel(
        functools.partial(_dot_head_kernel, d, n_chunks),
        out_type=jax.ShapeDtypeStruct((b,), jnp.float32),
        mesh=mesh,
        compiler_params=pltpu.CompilerParams(needs_layout_passes=False),
        scratch_types=[
            pltpu.VMEM((2, CHUNK), jnp.int32),
            pltpu.VMEM((2, CHUNK), jnp.int32),
            pltpu.VMEM((2, CHUNK, d), jnp.float32),
            pltpu.VMEM((2, CHUNK, d), jnp.float32),
            pltpu.VMEM((CHUNK, LANES), jnp.float32),
            pltpu.VMEM((2, CHUNK), jnp.float32),
            pltpu.SemaphoreType.DMA,
            pltpu.SemaphoreType.DMA,
            pltpu.SemaphoreType.DMA,
            pltpu.SemaphoreType.DMA,
            pltpu.SemaphoreType.DMA,
            pltpu.SemaphoreType.DMA,
        ],
    )
    return run(node_embeddings, src_idx, tgt_idx)
